# Initial kernel scaffold; baseline (speedup 1.0000x reference)
import jax
import jax.numpy as jnp
from jax.experimental import pallas as pl


def kernel(x, edge_index, W1, b1, W2, b2):
    raise NotImplementedError("write your pallas kernel here")


# speedup vs baseline: 7.2599x; 7.2599x over previous
import functools

import jax
import jax.numpy as jnp
from jax import lax
from jax.experimental import pallas as pl
from jax.experimental.pallas import tpu as pltpu
from jax.experimental.pallas import tpu_sc as plsc

N = 10000
E = 320000
D = 128
F = 64
W = 80
NC = 2
NS = 16
NW = NC * NS
B = 128
NB = 79
CHUNK = NB * B
EP = NW * CHUNK
ROWS = 10112
RPS = ROWS // NS
ZR = 79


def _sc_agg_body(tbl_hbm, idx_hbm, out_hbm, acc_s, src_v, dst_v, gbuf, zbuf):
    c = lax.axis_index("c")
    s = lax.axis_index("s")
    w = c * NS + s

    z16 = jnp.zeros((16,), jnp.float32)

    def zero_row(r, _):
        for cc in range(W // 16):
            zbuf[r, pl.ds(cc * 16, 16)] = z16
        return _

    lax.fori_loop(0, ZR, zero_row, None)

    def zero_chunk(k, _):
        pltpu.sync_copy(zbuf, acc_s.at[pl.ds(s * RPS + k * ZR, ZR), :])
        return _

    lax.fori_loop(0, RPS // ZR, zero_chunk, None)

    pltpu.sync_copy(idx_hbm.at[0, w], src_v)
    pltpu.sync_copy(idx_hbm.at[1, w], dst_v)

    plsc.subcore_barrier()

    def edge_batch(b, _):
        pltpu.sync_copy(tbl_hbm.at[dst_v.at[b]], gbuf)
        pltpu.sync_copy(gbuf, acc_s.at[src_v.at[b]], add=True)
        pltpu.sync_copy(tbl_hbm.at[src_v.at[b]], gbuf)
        pltpu.sync_copy(gbuf, acc_s.at[dst_v.at[b]], add=True)
        return _

    lax.fori_loop(0, NB, edge_batch, None)

    plsc.subcore_barrier()

    pltpu.sync_copy(acc_s.at[pl.ds(s * RPS, RPS), :],
                    out_hbm.at[c, pl.ds(s * RPS, RPS), :])


def _sc_aggregate(tbl, idx):
    mesh = plsc.VectorSubcoreMesh(core_axis_name="c", subcore_axis_name="s",
                                  num_cores=NC, num_subcores=NS)
    f = pl.kernel(
        _sc_agg_body,
        out_type=jax.ShapeDtypeStruct((NC, ROWS, W), jnp.float32),
        mesh=mesh,
        scratch_types=[
            pltpu.VMEM_SHARED((ROWS, W), jnp.float32),
            pltpu.VMEM((NB, B), jnp.int32),
            pltpu.VMEM((NB, B), jnp.int32),
            pltpu.VMEM((B, W), jnp.float32),
            pltpu.VMEM((ZR, W), jnp.float32),
        ],
        compiler_params=pltpu.CompilerParams(use_tc_tiling_on_sc=False),
    )
    return f(tbl, idx)


def _tc_mlp_body(x_ref, p0_ref, p1_ref, w1_ref, b1_ref, w2_ref, b2_ref, o_ref):
    p = p0_ref[...] + p1_ref[...]
    deg = jnp.maximum(p[:, F:F + 1], 1.0)
    agg = p[:, :F] / deg
    combined = jnp.concatenate([x_ref[:, :F], agg], axis=1)
    h = jnp.dot(combined, w1_ref[...],
                preferred_element_type=jnp.float32,
                precision=lax.Precision.HIGHEST) + b1_ref[...]
    h = jnp.maximum(h, 0.0)
    o_ref[...] = jnp.dot(h, w2_ref[...],
                         preferred_element_type=jnp.float32,
                         precision=lax.Precision.HIGHEST) + b2_ref[...]


def _tc_mlp(x, p0, p1, W1, b1, W2, b2):
    R = 1000
    grid = (N // R,)
    return pl.pallas_call(
        _tc_mlp_body,
        grid=grid,
        in_specs=[
            pl.BlockSpec((R, D), lambda i: (i, 0)),
            pl.BlockSpec((R, W), lambda i: (i, 0)),
            pl.BlockSpec((R, W), lambda i: (i, 0)),
            pl.BlockSpec((D, D), lambda i: (0, 0)),
            pl.BlockSpec((1, D), lambda i: (0, 0)),
            pl.BlockSpec((D, D), lambda i: (0, 0)),
            pl.BlockSpec((1, D), lambda i: (0, 0)),
        ],
        out_specs=pl.BlockSpec((R, D), lambda i: (i, 0)),
        out_shape=jax.ShapeDtypeStruct((N, D), jnp.float32),
    )(x, p0, p1, W1, b1.reshape(1, D), W2, b2.reshape(1, D))


def kernel(x, edge_index, W1, b1, W2, b2):
    tbl = jnp.zeros((ROWS, W), jnp.float32)
    tbl = tbl.at[:N, :F].set(x[:, F:])
    tbl = tbl.at[:N, F].set(1.0)

    pad = jnp.full((2, EP - E), N, dtype=jnp.int32)
    idx = jnp.concatenate([edge_index, pad], axis=1).reshape(2, NW, NB, B)

    parts = _sc_aggregate(tbl, idx)
    return _tc_mlp(x, parts[0, :N], parts[1, :N], W1, b1, W2, b2)


# speedup vs baseline: 8.9263x; 1.2295x over previous
import functools

import jax
import jax.numpy as jnp
from jax import lax
from jax.experimental import pallas as pl
from jax.experimental.pallas import tpu as pltpu
from jax.experimental.pallas import tpu_sc as plsc

N = 10000
E = 320000
D = 128
F = 64
W = 80
NC = 2
NS = 16
NW = NC * NS
B = 128
NB = 79
CHUNK = NB * B
EP = NW * CHUNK
ROWS = 10112
RPS = ROWS // NS
ZR = 79


def _sc_agg_body(tbl_hbm, idx_hbm, out_hbm, acc_s, src_v, dst_v,
                 gbuf0, gbuf1, gbuf2, gbuf3, zbuf, gsems, ssems):
    c = lax.axis_index("c")
    s = lax.axis_index("s")
    w = c * NS + s

    z16 = jnp.zeros((16,), jnp.float32)

    def zero_row(r, _):
        for cc in range(W // 16):
            zbuf[r, pl.ds(cc * 16, 16)] = z16
        return _

    lax.fori_loop(0, ZR, zero_row, None)

    def zero_chunk(k, _):
        pltpu.sync_copy(zbuf, acc_s.at[pl.ds(s * RPS + k * ZR, ZR), :])
        return _

    lax.fori_loop(0, RPS // ZR, zero_chunk, None)

    pltpu.sync_copy(idx_hbm.at[0, w], src_v)
    pltpu.sync_copy(idx_hbm.at[1, w], dst_v)

    plsc.subcore_barrier()

    gbufs = (gbuf0, gbuf1, gbuf2, gbuf3)

    def task_refs(u, k):
        b = 2 * k + (u >> 1)
        if u & 1 == 0:
            return dst_v.at[b], src_v.at[b]
        return src_v.at[b], dst_v.at[b]

    def issue_gather(u, k):
        g_idx, _ = task_refs(u, k)
        pltpu.async_copy(tbl_hbm.at[g_idx], gbufs[u], gsems.at[u])

    def wait_gather(u, k):
        g_idx, _ = task_refs(u, k)
        pltpu.make_async_copy(tbl_hbm.at[g_idx], gbufs[u], gsems.at[u]).wait()

    def issue_scatter(u, k):
        _, s_idx = task_refs(u, k)
        pltpu.async_copy(gbufs[u], acc_s.at[s_idx], ssems.at[u], add=True)

    def wait_scatter(u, k):
        _, s_idx = task_refs(u, k)
        pltpu.make_async_copy(gbufs[u], acc_s.at[s_idx], ssems.at[u]).wait()

    for u in range(4):
        issue_gather(u, 0)

    NG = (2 * NB) // 4

    def group(k, _):
        for u in range(4):
            wait_gather(u, k)
            issue_scatter(u, k)
        for u in range(4):
            wait_scatter(u, k)

        @pl.when(k < NG - 1)
        def _issue_next():
            for u in range(4):
                issue_gather(u, k + 1)

        return _

    lax.fori_loop(0, NG, group, None)

    pltpu.sync_copy(tbl_hbm.at[dst_v.at[NB - 1]], gbuf0)
    pltpu.sync_copy(gbuf0, acc_s.at[src_v.at[NB - 1]], add=True)
    pltpu.sync_copy(tbl_hbm.at[src_v.at[NB - 1]], gbuf1)
    pltpu.sync_copy(gbuf1, acc_s.at[dst_v.at[NB - 1]], add=True)

    plsc.subcore_barrier()

    pltpu.sync_copy(acc_s.at[pl.ds(s * RPS, RPS), :],
                    out_hbm.at[c, pl.ds(s * RPS, RPS), :])


def _sc_aggregate(tbl, idx):
    mesh = plsc.VectorSubcoreMesh(core_axis_name="c", subcore_axis_name="s",
                                  num_cores=NC, num_subcores=NS)
    f = pl.kernel(
        _sc_agg_body,
        out_type=jax.ShapeDtypeStruct((NC, ROWS, W), jnp.float32),
        mesh=mesh,
        scratch_types=[
            pltpu.VMEM_SHARED((ROWS, W), jnp.float32),
            pltpu.VMEM((NB, B), jnp.int32),
            pltpu.VMEM((NB, B), jnp.int32),
            pltpu.VMEM((B, W), jnp.float32),
            pltpu.VMEM((B, W), jnp.float32),
            pltpu.VMEM((B, W), jnp.float32),
            pltpu.VMEM((B, W), jnp.float32),
            pltpu.VMEM((ZR, W), jnp.float32),
            pltpu.SemaphoreType.DMA((4,)),
            pltpu.SemaphoreType.DMA((4,)),
        ],
        compiler_params=pltpu.CompilerParams(use_tc_tiling_on_sc=False),
    )
    return f(tbl, idx)


def _tc_mlp_body(x_ref, p0_ref, p1_ref, w1_ref, b1_ref, w2_ref, b2_ref, o_ref):
    p = p0_ref[...] + p1_ref[...]
    deg = jnp.maximum(p[:, F:F + 1], 1.0)
    agg = p[:, :F] / deg
    combined = jnp.concatenate([x_ref[:, :F], agg], axis=1)
    h = jnp.dot(combined, w1_ref[...],
                preferred_element_type=jnp.float32,
                precision=lax.Precision.HIGHEST) + b1_ref[...]
    h = jnp.maximum(h, 0.0)
    o_ref[...] = jnp.dot(h, w2_ref[...],
                         preferred_element_type=jnp.float32,
                         precision=lax.Precision.HIGHEST) + b2_ref[...]


def _tc_mlp(x, p0, p1, W1, b1, W2, b2):
    R = 1000
    grid = (N // R,)
    return pl.pallas_call(
        _tc_mlp_body,
        grid=grid,
        in_specs=[
            pl.BlockSpec((R, D), lambda i: (i, 0)),
            pl.BlockSpec((R, W), lambda i: (i, 0)),
            pl.BlockSpec((R, W), lambda i: (i, 0)),
            pl.BlockSpec((D, D), lambda i: (0, 0)),
            pl.BlockSpec((1, D), lambda i: (0, 0)),
            pl.BlockSpec((D, D), lambda i: (0, 0)),
            pl.BlockSpec((1, D), lambda i: (0, 0)),
        ],
        out_specs=pl.BlockSpec((R, D), lambda i: (i, 0)),
        out_shape=jax.ShapeDtypeStruct((N, D), jnp.float32),
    )(x, p0, p1, W1, b1.reshape(1, D), W2, b2.reshape(1, D))


def kernel(x, edge_index, W1, b1, W2, b2):
    tbl = jnp.zeros((ROWS, W), jnp.float32)
    tbl = tbl.at[:N, :F].set(x[:, F:])
    tbl = tbl.at[:N, F].set(1.0)

    pad = jnp.full((2, EP - E), N, dtype=jnp.int32)
    idx = jnp.concatenate([edge_index, pad], axis=1).reshape(2, NW, NB, B)

    parts = _sc_aggregate(tbl, idx)
    return _tc_mlp(x, parts[0, :N], parts[1, :N], W1, b1, W2, b2)


# speedup vs baseline: 9.5752x; 1.0727x over previous
import functools

import jax
import jax.numpy as jnp
from jax import lax
from jax.experimental import pallas as pl
from jax.experimental.pallas import tpu as pltpu
from jax.experimental.pallas import tpu_sc as plsc

N = 10000
E = 320000
D = 128
F = 64
W = 72
NC = 2
NS = 16
NW = NC * NS
B = 128
NB = 79
CHUNK = NB * B
EP = NW * CHUNK
ROWS = 10112
RPS = ROWS // NS
ZR = 79


def _sc_agg_body(tbl_hbm, idx_hbm, out_hbm, acc_s, src_v, dst_v,
                 gbuf0, gbuf1, gbuf2, gbuf3, zbuf, gsems, ssems):
    c = lax.axis_index("c")
    s = lax.axis_index("s")
    w = c * NS + s

    z16 = jnp.zeros((16,), jnp.float32)

    def zero_row(r, _):
        for cc in range(W // 16):
            zbuf[r, pl.ds(cc * 16, 16)] = z16
        return _

    lax.fori_loop(0, ZR, zero_row, None)

    def zero_chunk(k, _):
        pltpu.sync_copy(zbuf, acc_s.at[pl.ds(s * RPS + k * ZR, ZR), :])
        return _

    lax.fori_loop(0, RPS // ZR, zero_chunk, None)

    pltpu.sync_copy(idx_hbm.at[0, w], src_v)
    pltpu.sync_copy(idx_hbm.at[1, w], dst_v)

    plsc.subcore_barrier()

    gbufs = (gbuf0, gbuf1, gbuf2, gbuf3)

    def task_refs(u, k):
        b = 2 * k + (u >> 1)
        if u & 1 == 0:
            return dst_v.at[b], src_v.at[b]
        return src_v.at[b], dst_v.at[b]

    def issue_gather(u, k):
        g_idx, _ = task_refs(u, k)
        pltpu.async_copy(tbl_hbm.at[g_idx], gbufs[u], gsems.at[u])

    def wait_gather(u, k):
        g_idx, _ = task_refs(u, k)
        pltpu.make_async_copy(tbl_hbm.at[g_idx], gbufs[u], gsems.at[u]).wait()

    def issue_scatter(u, k):
        _, s_idx = task_refs(u, k)
        pltpu.async_copy(gbufs[u], acc_s.at[s_idx], ssems.at[u], add=True)

    def wait_scatter(u, k):
        _, s_idx = task_refs(u, k)
        pltpu.make_async_copy(gbufs[u], acc_s.at[s_idx], ssems.at[u]).wait()

    for u in range(4):
        issue_gather(u, 0)

    NG = (2 * NB) // 4

    def group(k, _):
        for u in range(4):
            wait_gather(u, k)
            issue_scatter(u, k)
        for u in range(4):
            wait_scatter(u, k)

        @pl.when(k < NG - 1)
        def _issue_next():
            for u in range(4):
                issue_gather(u, k + 1)

        return _

    lax.fori_loop(0, NG, group, None)

    pltpu.sync_copy(tbl_hbm.at[dst_v.at[NB - 1]], gbuf0)
    pltpu.sync_copy(gbuf0, acc_s.at[src_v.at[NB - 1]], add=True)
    pltpu.sync_copy(tbl_hbm.at[src_v.at[NB - 1]], gbuf1)
    pltpu.sync_copy(gbuf1, acc_s.at[dst_v.at[NB - 1]], add=True)

    plsc.subcore_barrier()

    pltpu.sync_copy(acc_s.at[pl.ds(s * RPS, RPS), :],
                    out_hbm.at[c, pl.ds(s * RPS, RPS), :])


def _sc_aggregate(tbl, idx):
    mesh = plsc.VectorSubcoreMesh(core_axis_name="c", subcore_axis_name="s",
                                  num_cores=NC, num_subcores=NS)
    f = pl.kernel(
        _sc_agg_body,
        out_type=jax.ShapeDtypeStruct((NC, ROWS, W), jnp.float32),
        mesh=mesh,
        scratch_types=[
            pltpu.VMEM_SHARED((ROWS, W), jnp.float32),
            pltpu.VMEM((NB, B), jnp.int32),
            pltpu.VMEM((NB, B), jnp.int32),
            pltpu.VMEM((B, W), jnp.float32),
            pltpu.VMEM((B, W), jnp.float32),
            pltpu.VMEM((B, W), jnp.float32),
            pltpu.VMEM((B, W), jnp.float32),
            pltpu.VMEM((ZR, W), jnp.float32),
            pltpu.SemaphoreType.DMA((4,)),
            pltpu.SemaphoreType.DMA((4,)),
        ],
        compiler_params=pltpu.CompilerParams(use_tc_tiling_on_sc=False),
    )
    return f(tbl, idx)


def _tc_pre_body(x_ref, w1a_ref, b1_ref, o_ref):
    o_ref[...] = jnp.dot(x_ref[:, :F], w1a_ref[...],
                         preferred_element_type=jnp.float32,
                         precision=lax.Precision.HIGHEST) + b1_ref[...]


def _tc_pre(x, W1, b1):
    R = 1000
    return pl.pallas_call(
        _tc_pre_body,
        grid=(N // R,),
        in_specs=[
            pl.BlockSpec((R, D), lambda i: (i, 0)),
            pl.BlockSpec((F, D), lambda i: (0, 0)),
            pl.BlockSpec((1, D), lambda i: (0, 0)),
        ],
        out_specs=pl.BlockSpec((R, D), lambda i: (i, 0)),
        out_shape=jax.ShapeDtypeStruct((N, D), jnp.float32),
    )(x, W1[:F], b1.reshape(1, D))


def _tc_mlp_body(pre_ref, p0_ref, p1_ref, w1b_ref, w2_ref, b2_ref, o_ref):
    p = p0_ref[...] + p1_ref[...]
    deg = jnp.maximum(p[:, F:F + 1], 1.0)
    agg = p[:, :F] / deg
    h = pre_ref[...] + jnp.dot(agg, w1b_ref[...],
                               preferred_element_type=jnp.float32,
                               precision=lax.Precision.HIGHEST)
    h = jnp.maximum(h, 0.0)
    o_ref[...] = jnp.dot(h, w2_ref[...],
                         preferred_element_type=jnp.float32,
                         precision=lax.Precision.HIGHEST) + b2_ref[...]


def _tc_mlp(pre, p0, p1, W1, W2, b2):
    R = 1000
    return pl.pallas_call(
        _tc_mlp_body,
        grid=(N // R,),
        in_specs=[
            pl.BlockSpec((R, D), lambda i: (i, 0)),
            pl.BlockSpec((R, W), lambda i: (i, 0)),
            pl.BlockSpec((R, W), lambda i: (i, 0)),
            pl.BlockSpec((F, D), lambda i: (0, 0)),
            pl.BlockSpec((D, D), lambda i: (0, 0)),
            pl.BlockSpec((1, D), lambda i: (0, 0)),
        ],
        out_specs=pl.BlockSpec((R, D), lambda i: (i, 0)),
        out_shape=jax.ShapeDtypeStruct((N, D), jnp.float32),
    )(pre, p0, p1, W1[F:], W2, b2.reshape(1, D))


def kernel(x, edge_index, W1, b1, W2, b2):
    tbl = jnp.zeros((ROWS, W), jnp.float32)
    tbl = tbl.at[:N, :F].set(x[:, F:])
    tbl = tbl.at[:N, F].set(1.0)

    pad = jnp.full((2, EP - E), N, dtype=jnp.int32)
    idx = jnp.concatenate([edge_index, pad], axis=1).reshape(2, NW, NB, B)

    tbl, idx = jax.lax.optimization_barrier((tbl, idx))

    pre = _tc_pre(x, W1, b1)
    parts = _sc_aggregate(tbl, idx)
    return _tc_mlp(pre, parts[0, :N], parts[1, :N], W1, W2, b2)


# speedup vs baseline: 16.7138x; 1.7455x over previous
import functools

import jax
import jax.numpy as jnp
from jax import lax
from jax.experimental import pallas as pl
from jax.experimental.pallas import tpu as pltpu
from jax.experimental.pallas import tpu_sc as plsc

N = 10000
E = 320000
D = 128
F = 64
W = 72
NC = 2
NS = 16
NW = NC * NS
B = 128
NB = 79
CHUNK = NB * B
EP = NW * CHUNK
ROWS = 10112
RPS = ROWS // NS
ZR = 79


def _sc_agg_body(tbl_hbm, idx_hbm, out_hbm, acc_s, src_v, dst_v,
                 gbuf0, gbuf1, gbuf2, gbuf3, zbuf, gsems, ssems):
    c = lax.axis_index("c")
    s = lax.axis_index("s")
    w = c * NS + s

    z16 = jnp.zeros((16,), jnp.float32)

    def zero_row(r, _):
        for cc in range(W // 16):
            zbuf[r, pl.ds(cc * 16, 16)] = z16
        return _

    lax.fori_loop(0, ZR, zero_row, None)

    def zero_chunk(k, _):
        pltpu.sync_copy(zbuf, acc_s.at[pl.ds(s * RPS + k * ZR, ZR), :])
        return _

    lax.fori_loop(0, RPS // ZR, zero_chunk, None)

    pltpu.sync_copy(idx_hbm.at[0, w], src_v)
    pltpu.sync_copy(idx_hbm.at[1, w], dst_v)

    plsc.subcore_barrier()

    gbufs = (gbuf0, gbuf1, gbuf2, gbuf3)

    def task_refs(u, k):
        b = 2 * k + (u >> 1)
        if u & 1 == 0:
            return dst_v.at[b], src_v.at[b]
        return src_v.at[b], dst_v.at[b]

    def issue_gather(u, k):
        g_idx, _ = task_refs(u, k)
        pltpu.async_copy(tbl_hbm.at[g_idx], gbufs[u], gsems.at[u])

    def wait_gather(u, k):
        g_idx, _ = task_refs(u, k)
        pltpu.make_async_copy(tbl_hbm.at[g_idx], gbufs[u], gsems.at[u]).wait()

    def issue_scatter(u, k):
        _, s_idx = task_refs(u, k)
        pltpu.async_copy(gbufs[u], acc_s.at[s_idx], ssems.at[u], add=True)

    def wait_scatter(u, k):
        _, s_idx = task_refs(u, k)
        pltpu.make_async_copy(gbufs[u], acc_s.at[s_idx], ssems.at[u]).wait()

    for u in range(4):
        issue_gather(u, 0)

    NG = (2 * NB) // 4

    def group(k, _):
        for u in range(4):
            wait_gather(u, k)
            issue_scatter(u, k)
        for u in range(4):
            wait_scatter(u, k)

        @pl.when(k < NG - 1)
        def _issue_next():
            for u in range(4):
                issue_gather(u, k + 1)

        return _

    lax.fori_loop(0, NG, group, None)

    pltpu.sync_copy(tbl_hbm.at[dst_v.at[NB - 1]], gbuf0)
    pltpu.sync_copy(gbuf0, acc_s.at[src_v.at[NB - 1]], add=True)
    pltpu.sync_copy(tbl_hbm.at[src_v.at[NB - 1]], gbuf1)
    pltpu.sync_copy(gbuf1, acc_s.at[dst_v.at[NB - 1]], add=True)

    plsc.subcore_barrier()

    pltpu.sync_copy(acc_s.at[pl.ds(s * RPS, RPS), :],
                    out_hbm.at[c, pl.ds(s * RPS, RPS), :])


def _sc_aggregate(tbl, idx):
    mesh = plsc.VectorSubcoreMesh(core_axis_name="c", subcore_axis_name="s",
                                  num_cores=NC, num_subcores=NS)
    f = pl.kernel(
        _sc_agg_body,
        out_type=jax.ShapeDtypeStruct((NC, ROWS, W), jnp.float32),
        mesh=mesh,
        scratch_types=[
            pltpu.VMEM_SHARED((ROWS, W), jnp.float32),
            pltpu.VMEM((NB, B), jnp.int32),
            pltpu.VMEM((NB, B), jnp.int32),
            pltpu.VMEM((B, W), jnp.float32),
            pltpu.VMEM((B, W), jnp.float32),
            pltpu.VMEM((B, W), jnp.float32),
            pltpu.VMEM((B, W), jnp.float32),
            pltpu.VMEM((ZR, W), jnp.float32),
            pltpu.SemaphoreType.DMA((4,)),
            pltpu.SemaphoreType.DMA((4,)),
        ],
        compiler_params=pltpu.CompilerParams(use_tc_tiling_on_sc=False),
    )
    return f(tbl, idx)


def _tc_pre_body(x_ref, w1a_ref, b1_ref, o_ref):
    o_ref[...] = jnp.dot(x_ref[:, :F], w1a_ref[...],
                         preferred_element_type=jnp.float32,
                         precision=lax.Precision.HIGHEST) + b1_ref[...]


def _tc_pre(x, W1, b1):
    R = 1000
    return pl.pallas_call(
        _tc_pre_body,
        grid=(N // R,),
        in_specs=[
            pl.BlockSpec((R, D), lambda i: (i, 0)),
            pl.BlockSpec((F, D), lambda i: (0, 0)),
            pl.BlockSpec((1, D), lambda i: (0, 0)),
        ],
        out_specs=pl.BlockSpec((R, D), lambda i: (i, 0)),
        out_shape=jax.ShapeDtypeStruct((N, D), jnp.float32),
    )(x, W1[:F], b1.reshape(1, D))


def _tc_mlp_body(pre_ref, p0_ref, p1_ref, w1b_ref, w2_ref, b2_ref, o_ref):
    p = p0_ref[...] + p1_ref[...]
    deg = jnp.maximum(p[:, F:F + 1], 1.0)
    agg = p[:, :F] / deg
    h = pre_ref[...] + jnp.dot(agg, w1b_ref[...],
                               preferred_element_type=jnp.float32,
                               precision=lax.Precision.HIGHEST)
    h = jnp.maximum(h, 0.0)
    o_ref[...] = jnp.dot(h, w2_ref[...],
                         preferred_element_type=jnp.float32,
                         precision=lax.Precision.HIGHEST) + b2_ref[...]


def _tc_mlp(pre, p0, p1, W1, W2, b2):
    R = 1000
    return pl.pallas_call(
        _tc_mlp_body,
        grid=(N // R,),
        in_specs=[
            pl.BlockSpec((R, D), lambda i: (i, 0)),
            pl.BlockSpec((R, W), lambda i: (i, 0)),
            pl.BlockSpec((R, W), lambda i: (i, 0)),
            pl.BlockSpec((F, D), lambda i: (0, 0)),
            pl.BlockSpec((D, D), lambda i: (0, 0)),
            pl.BlockSpec((1, D), lambda i: (0, 0)),
        ],
        out_specs=pl.BlockSpec((R, D), lambda i: (i, 0)),
        out_shape=jax.ShapeDtypeStruct((N, D), jnp.float32),
    )(pre, p0, p1, W1[F:], W2, b2.reshape(1, D))


def kernel(x, edge_index, W1, b1, W2, b2):
    tbl = jnp.zeros((ROWS, W), jnp.float32)
    tbl = tbl.at[:N, :F].set(x[:, F:])
    tbl = tbl.at[:N, F].set(1.0)

    pad_row = N + (jnp.arange(EP - E, dtype=jnp.int32) % (ROWS - N))
    pad = jnp.stack([pad_row, pad_row])
    idx = jnp.concatenate([edge_index, pad], axis=1).reshape(2, NW, NB, B)

    tbl, idx = jax.lax.optimization_barrier((tbl, idx))

    pre = _tc_pre(x, W1, b1)
    parts = _sc_aggregate(tbl, idx)
    return _tc_mlp(pre, parts[0, :N], parts[1, :N], W1, W2, b2)


# speedup vs baseline: 20.1938x; 1.2082x over previous
import jax
import jax.numpy as jnp
from jax import lax
from jax.experimental import pallas as pl
from jax.experimental.pallas import tpu as pltpu
from jax.experimental.pallas import tpu_sc as plsc

N = 10000
E = 320000
D = 128
F = 64
W = 72
NC = 2
NS = 16
NW = NC * NS
B = 80
NB = 125
NRING = 8
ROWS = 10112
RPS = ROWS // NS
ZR = 79
RT = 1000


def _sc_agg_body(tbl_hbm, idx_hbm, out_hbm, acc_s, src_v, dst_v,
                 gbuf0, gbuf1, gbuf2, gbuf3, gbuf4, gbuf5, gbuf6, gbuf7,
                 zbuf, gsems, ssems):
    c = lax.axis_index("c")
    s = lax.axis_index("s")
    w = c * NS + s

    z16 = jnp.zeros((16,), jnp.float32)

    def zero_row(r, _):
        for cc in range(W // 16):
            zbuf[r, pl.ds(cc * 16, 16)] = z16
        return _

    lax.fori_loop(0, ZR, zero_row, None)

    def zero_chunk(k, _):
        pltpu.sync_copy(zbuf, acc_s.at[pl.ds(s * RPS + k * ZR, ZR), :])
        return _

    lax.fori_loop(0, RPS // ZR, zero_chunk, None)

    pltpu.sync_copy(idx_hbm.at[0, w], src_v)
    pltpu.sync_copy(idx_hbm.at[1, w], dst_v)

    plsc.subcore_barrier()

    gbufs = (gbuf0, gbuf1, gbuf2, gbuf3, gbuf4, gbuf5, gbuf6, gbuf7)

    def task_refs(u, k):
        b = (NRING // 2) * k + (u >> 1)
        if u & 1 == 0:
            return dst_v.at[b], src_v.at[b]
        return src_v.at[b], dst_v.at[b]

    def issue_gather(u, k):
        g_idx, _ = task_refs(u, k)
        pltpu.async_copy(tbl_hbm.at[g_idx], gbufs[u], gsems.at[u])

    def wait_gather(u, k):
        g_idx, _ = task_refs(u, k)
        pltpu.make_async_copy(tbl_hbm.at[g_idx], gbufs[u], gsems.at[u]).wait()

    def issue_scatter(u, k):
        _, s_idx = task_refs(u, k)
        pltpu.async_copy(gbufs[u], acc_s.at[s_idx], ssems.at[u], add=True)

    def wait_scatter(u, k):
        _, s_idx = task_refs(u, k)
        pltpu.make_async_copy(gbufs[u], acc_s.at[s_idx], ssems.at[u]).wait()

    for u in range(NRING):
        issue_gather(u, 0)

    NG = (2 * NB) // NRING

    def group(k, _):
        for u in range(NRING):
            wait_gather(u, k)
            issue_scatter(u, k)
        for u in range(NRING):
            wait_scatter(u, k)

        @pl.when(k < NG - 1)
        def _issue_next():
            for u in range(NRING):
                issue_gather(u, k + 1)

        return _

    lax.fori_loop(0, NG, group, None)

    pltpu.sync_copy(tbl_hbm.at[dst_v.at[NB - 1]], gbuf0)
    pltpu.sync_copy(gbuf0, acc_s.at[src_v.at[NB - 1]], add=True)
    pltpu.sync_copy(tbl_hbm.at[src_v.at[NB - 1]], gbuf1)
    pltpu.sync_copy(gbuf1, acc_s.at[dst_v.at[NB - 1]], add=True)

    plsc.subcore_barrier()

    pltpu.sync_copy(acc_s.at[pl.ds(s * RPS, RPS), :],
                    out_hbm.at[c, pl.ds(s * RPS, RPS), :])


def _sc_aggregate(tbl, idx):
    mesh = plsc.VectorSubcoreMesh(core_axis_name="c", subcore_axis_name="s",
                                  num_cores=NC, num_subcores=NS)
    f = pl.kernel(
        _sc_agg_body,
        out_type=jax.ShapeDtypeStruct((NC, ROWS, W), jnp.float32),
        mesh=mesh,
        scratch_types=[
            pltpu.VMEM_SHARED((ROWS, W), jnp.float32),
            pltpu.VMEM((NB, B), jnp.int32),
            pltpu.VMEM((NB, B), jnp.int32),
        ] + [pltpu.VMEM((B, W), jnp.float32)] * NRING + [
            pltpu.VMEM((ZR, W), jnp.float32),
            pltpu.SemaphoreType.DMA((NRING,)),
            pltpu.SemaphoreType.DMA((NRING,)),
        ],
        compiler_params=pltpu.CompilerParams(use_tc_tiling_on_sc=False),
    )
    return f(tbl, idx)


def _tc_pre_body(x_ref, w1a_ref, b1_ref, pre_ref, tbl_ref):
    pre_ref[...] = jnp.dot(x_ref[:, :F], w1a_ref[...],
                           preferred_element_type=jnp.float32,
                           precision=lax.Precision.HIGHEST) + b1_ref[...]
    tbl_ref[:, :F] = x_ref[:, F:]
    one_col = (lax.broadcasted_iota(jnp.int32, (RT, W - F), 1) == 0)
    tbl_ref[:, F:] = one_col.astype(jnp.float32)


def _tc_pre(x, W1, b1):
    return pl.pallas_call(
        _tc_pre_body,
        grid=(N // RT,),
        in_specs=[
            pl.BlockSpec((RT, D), lambda i: (i, 0)),
            pl.BlockSpec((F, D), lambda i: (0, 0)),
            pl.BlockSpec((1, D), lambda i: (0, 0)),
        ],
        out_specs=[
            pl.BlockSpec((RT, D), lambda i: (i, 0)),
            pl.BlockSpec((RT, W), lambda i: (i, 0)),
        ],
        out_shape=[
            jax.ShapeDtypeStruct((N, D), jnp.float32),
            jax.ShapeDtypeStruct((N, W), jnp.float32),
        ],
    )(x, W1[:F], b1.reshape(1, D))


def _tc_mlp_body(pre_ref, p0_ref, p1_ref, w1b_ref, w2_ref, b2_ref, o_ref):
    p = p0_ref[0] + p1_ref[0]
    deg = jnp.maximum(p[:, F:F + 1], 1.0)
    agg = p[:, :F] / deg
    h = pre_ref[...] + jnp.dot(agg, w1b_ref[...],
                               preferred_element_type=jnp.float32,
                               precision=lax.Precision.HIGHEST)
    h = jnp.maximum(h, 0.0)
    o_ref[...] = jnp.dot(h, w2_ref[...],
                         preferred_element_type=jnp.float32,
                         precision=lax.Precision.HIGHEST) + b2_ref[...]


def _tc_mlp(pre, parts, W1, W2, b2):
    return pl.pallas_call(
        _tc_mlp_body,
        grid=(N // RT,),
        in_specs=[
            pl.BlockSpec((RT, D), lambda i: (i, 0)),
            pl.BlockSpec((1, RT, W), lambda i: (0, i, 0)),
            pl.BlockSpec((1, RT, W), lambda i: (1, i, 0)),
            pl.BlockSpec((F, D), lambda i: (0, 0)),
            pl.BlockSpec((D, D), lambda i: (0, 0)),
            pl.BlockSpec((1, D), lambda i: (0, 0)),
        ],
        out_specs=pl.BlockSpec((RT, D), lambda i: (i, 0)),
        out_shape=jax.ShapeDtypeStruct((N, D), jnp.float32),
    )(pre, parts, parts, W1[F:], W2, b2.reshape(1, D))


def kernel(x, edge_index, W1, b1, W2, b2):
    idx = edge_index.reshape(2, NW, NB, B)
    pre, tbl = _tc_pre(x, W1, b1)
    parts = _sc_aggregate(tbl, idx)
    return _tc_mlp(pre, parts, W1, W2, b2)
